# baseline (speedup 1.0000x reference)
import dataclasses
import functools

import jax
import jax.numpy as jnp
from jax import lax
from jax.experimental import pallas as pl
from jax.experimental.pallas import tpu as pltpu
from jax.experimental.pallas import tpu_sc as plsc

N = 10000
E = 320000
REL = 500
EH = 128

NW = 32
EPW = E // NW
CH = 80
NCH = EPW // CH


def _prep_body(rel_ref, ww_ref, wrel_ref, r2_ref, ro_ref):
    r = rel_ref[...]
    dn = (((1,), (0,)), ((), ()))
    r2_ref[...] = lax.dot_general(r, ww_ref[...], dn,
                                  precision=lax.Precision.HIGHEST,
                                  preferred_element_type=jnp.float32)
    ro_ref[...] = lax.dot_general(r, wrel_ref[...], dn,
                                  precision=lax.Precision.HIGHEST,
                                  preferred_element_type=jnp.float32)


_prep = pl.pallas_call(
    _prep_body,
    out_shape=[jax.ShapeDtypeStruct((REL, EH), jnp.float32),
               jax.ShapeDtypeStruct((REL, EH), jnp.float32)],
)


def _pmat_body(x_ref, r2_ref, p_ref):
    dnt = (((1,), (1,)), ((), ()))
    p_ref[...] = lax.dot_general(x_ref[...], r2_ref[...], dnt,
                                 precision=lax.Precision.HIGHEST,
                                 preferred_element_type=jnp.float32)


_pmat = pl.pallas_call(
    _pmat_body,
    grid=(5,),
    in_specs=[pl.BlockSpec((N // 5, EH), lambda i: (i, 0)),
              pl.BlockSpec((REL, EH), lambda i: (0, 0))],
    out_specs=pl.BlockSpec((N // 5, REL), lambda i: (i, 0)),
    out_shape=jax.ShapeDtypeStruct((N, REL), jnp.float32),
)


def _pidx_body(dst_ref, et_ref, pi_ref):
    pi_ref[...] = dst_ref[...] * REL + et_ref[...]


_pidx = pl.pallas_call(
    _pidx_body,
    out_shape=jax.ShapeDtypeStruct((NW, NCH, CH), jnp.int32),
)


def _softmax_body(dp_ref, w_ref):
    d = dp_ref[...]
    m = jnp.max(d)
    e = jnp.exp(d - m)
    w_ref[...] = e / jnp.sum(e)


_softmax = pl.pallas_call(
    _softmax_body,
    out_shape=jax.ShapeDtypeStruct((NW, NCH, CH), jnp.float32),
)


def _final_body(p_ref, out_ref):
    out_ref[...] = jnp.maximum(p_ref[0] + p_ref[1], 0.0)


_final = pl.pallas_call(
    _final_body,
    out_shape=jax.ShapeDtypeStruct((N, EH), jnp.float32),
)


@functools.cache
def _sc_kernels():
    mesh = plsc.VectorSubcoreMesh(core_axis_name="c", subcore_axis_name="s")
    cp = pltpu.CompilerParams()
    if "needs_layout_passes" in pltpu.CompilerParams.__dataclass_fields__:
        cp = dataclasses.replace(cp, needs_layout_passes=False)

    @functools.partial(
        pl.kernel,
        out_type=jax.ShapeDtypeStruct((NW, NCH, CH), jnp.float32),
        mesh=mesh,
        compiler_params=cp,
        scratch_types=[
            pltpu.VMEM((NCH, 3, CH), jnp.int32),
            pltpu.VMEM((CH, EH), jnp.float32),
            pltpu.VMEM((CH, EH), jnp.float32),
            pltpu.VMEM((CH,), jnp.float32),
            pltpu.VMEM((CH, EH), jnp.float32),
            pltpu.VMEM((CH, EH), jnp.float32),
            pltpu.VMEM((CH,), jnp.float32),
            pltpu.VMEM((16, 17), jnp.float32),
            pltpu.VMEM((NCH, CH), jnp.float32),
            pltpu.SemaphoreType.DMA,
            pltpu.SemaphoreType.DMA,
            pltpu.SemaphoreType.DMA,
            pltpu.SemaphoreType.DMA,
            pltpu.SemaphoreType.DMA,
            pltpu.SemaphoreType.DMA,
        ],
    )
    def _pass1(x_hbm, pf_hbm, pk_hbm, dp_hbm,
               pk_v, xsa_v, xda_v, pva_v, xsb_v, xdb_v, pvb_v, tb_v, dp_v,
               sa0, sa1, sa2, sb0, sb1, sb2):
        cid = lax.axis_index("c")
        sid = lax.axis_index("s")
        wid = cid * 16 + sid
        pltpu.sync_copy(pk_hbm.at[wid], pk_v)
        lanes = lax.iota(jnp.int32, 16)

        def fire(k, xs_v, xd_v, pv_v, s0, s1, s2):
            c1 = pltpu.async_copy(x_hbm.at[pk_v.at[k, 0]], xs_v, s0)
            c2 = pltpu.async_copy(x_hbm.at[pk_v.at[k, 1]], xd_v, s1)
            c3 = pltpu.async_copy(pf_hbm.at[pk_v.at[k, 2]], pv_v, s2)
            return (c1, c2, c3)

        def compute(k, xs_v, xd_v, pv_v):
            @pl.loop(0, CH // 16)
            def _grp(g):
                e0 = g * 16
                for j in range(16):
                    b = e0 + j
                    acc = xs_v[b, pl.ds(0, 16)] * xd_v[b, pl.ds(0, 16)]
                    for c in range(1, 8):
                        sl = pl.ds(c * 16, 16)
                        acc = acc + xs_v[b, sl] * xd_v[b, sl]
                    tb_v[j, pl.ds(0, 16)] = acc
                dpv = pv_v[pl.ds(e0, 16)]
                for j2 in range(16):
                    jv = jnp.full((16,), j2, jnp.int32)
                    dpv = dpv + plsc.load_gather(tb_v, [lanes, jv])
                dp_v[k, pl.ds(e0, 16)] = dpv

        ca = fire(0, xsa_v, xda_v, pva_v, sa0, sa1, sa2)
        cb = fire(1, xsb_v, xdb_v, pvb_v, sb0, sb1, sb2)
        del ca, cb

        @pl.loop(0, NCH // 2)
        def _pair(i):
            k0 = i * 2
            k1 = k0 + 1
            pltpu.make_async_copy(x_hbm.at[pk_v.at[k0, 0]], xsa_v, sa0).wait()
            pltpu.make_async_copy(x_hbm.at[pk_v.at[k0, 1]], xda_v, sa1).wait()
            pltpu.make_async_copy(pf_hbm.at[pk_v.at[k0, 2]], pva_v, sa2).wait()
            compute(k0, xsa_v, xda_v, pva_v)
            fire(k0 + 2, xsa_v, xda_v, pva_v, sa0, sa1, sa2)
            pltpu.make_async_copy(x_hbm.at[pk_v.at[k1, 0]], xsb_v, sb0).wait()
            pltpu.make_async_copy(x_hbm.at[pk_v.at[k1, 1]], xdb_v, sb1).wait()
            pltpu.make_async_copy(pf_hbm.at[pk_v.at[k1, 2]], pvb_v, sb2).wait()
            compute(k1, xsb_v, xdb_v, pvb_v)

            @pl.when(k1 + 2 < NCH)
            def _fb():
                fire(k1 + 2, xsb_v, xdb_v, pvb_v, sb0, sb1, sb2)

        pltpu.make_async_copy(x_hbm.at[pk_v.at[NCH - 1, 0]], xsa_v, sa0).wait()
        pltpu.make_async_copy(x_hbm.at[pk_v.at[NCH - 1, 1]], xda_v, sa1).wait()
        pltpu.make_async_copy(pf_hbm.at[pk_v.at[NCH - 1, 2]], pva_v, sa2).wait()
        compute(NCH - 1, xsa_v, xda_v, pva_v)

        pltpu.sync_copy(dp_v, dp_hbm.at[wid])

    @functools.partial(
        pl.kernel,
        out_type=jax.ShapeDtypeStruct((2, N, EH), jnp.float32),
        mesh=mesh,
        compiler_params=cp,
        scratch_types=[
            pltpu.VMEM((4, CH), jnp.int32),
            pltpu.VMEM((4, CH), jnp.int32),
            pltpu.VMEM((4, CH), jnp.int32),
            pltpu.VMEM((4, CH), jnp.int32),
            pltpu.VMEM((CH, EH), jnp.float32),
            pltpu.VMEM((CH, EH), jnp.float32),
            pltpu.VMEM((CH, EH), jnp.float32),
            pltpu.VMEM((CH, EH), jnp.float32),
            pltpu.VMEM_SHARED((N, EH), jnp.float32),
            pltpu.SemaphoreType.DMA,
            pltpu.SemaphoreType.DMA,
            pltpu.SemaphoreType.DMA,
            pltpu.SemaphoreType.DMA,
            pltpu.SemaphoreType.DMA,
            pltpu.SemaphoreType.DMA,
            pltpu.SemaphoreType.DMA,
            pltpu.SemaphoreType.DMA,
            pltpu.SemaphoreType.DMA,
            pltpu.SemaphoreType.DMA,
        ],
    )
    def _pass2(x_hbm, r2_hbm, pk_hbm, out_hbm,
               pka0, pkb0, pka1, pkb1, xsa_v, rra_v, xsb_v, rrb_v, acc_sh,
               spa0, spb0, spa1, spb1, sxa, sra, sxb, srb, sca, scb):
        cid = lax.axis_index("c")
        sid = lax.axis_index("s")
        wid = cid * 16 + sid
        zero16 = jnp.zeros((16,), jnp.float32)

        @pl.loop(0, CH)
        def _zrow(i):
            for c in range(8):
                xsa_v[i, pl.ds(c * 16, 16)] = zero16

        row0 = pl.multiple_of(sid * 624, 16)
        for q in range(7):
            pltpu.sync_copy(xsa_v, acc_sh.at[pl.ds(row0 + q * CH, CH)])
        pltpu.sync_copy(xsa_v.at[pl.ds(0, 64)],
                        acc_sh.at[pl.ds(row0 + 560, 64)])

        @pl.when(sid == 15)
        def _ztail():
            pltpu.sync_copy(xsa_v.at[pl.ds(0, 16)],
                            acc_sh.at[pl.ds(9984, 16)])

        plsc.subcore_barrier()

        ROWB = CH * EH * 4

        def fire_pk(k, pk_v, sem):
            pltpu.async_copy(pk_hbm.at[wid, k], pk_v, sem)

        def wait_pk(k, pk_v, sem):
            pltpu.make_async_copy(pk_hbm.at[wid, k], pk_v, sem).wait()

        def fire_xs(pk_v, xs_v, sem):
            pltpu.async_copy(x_hbm.at[pk_v.at[0]], xs_v, sem)

        def fire_rr(pk_v, rr_v, sem):
            pltpu.async_copy(r2_hbm.at[pk_v.at[2]], rr_v, sem)

        three = jnp.full((16,), 3, jnp.int32)

        def compute(pk_v, xs_v, rr_v):
            @pl.loop(0, CH, step=4)
            def _edge(b0):
                for j in range(4):
                    b = b0 + j
                    wi = plsc.load_gather(
                        pk_v, [three, jnp.full((16,), b, jnp.int32)])
                    wvec = plsc.bitcast(wi, jnp.float32)
                    for c in range(8):
                        sl = pl.ds(c * 16, 16)
                        rr_v[b, sl] = (xs_v[b, sl] + rr_v[b, sl]) * wvec

        def wait_sc(sem):
            pltpu.make_async_copy(rra_v, acc_sh.at[pka0.at[1]], sem).wait()

        fire_pk(0, pka0, spa0)
        fire_pk(1, pkb0, spb0)
        fire_xs(pka0, xsa_v, sxa)
        fire_rr(pka0, rra_v, sra)
        fire_xs(pkb0, xsb_v, sxb)

        @pl.loop(0, NCH // 4)
        def _body(i):
            k = i * 4

            def run(kk, pk_cur, sp_cur, xs_v, rr_v, sxP, srP, scP,
                    pk_fill, sp_fill, fill_ok,
                    pk_next, rrQ_v, srQ, scQ, first_ok=None):
                wait_pk(kk, pk_cur, sp_cur)
                pltpu.make_async_copy(
                    x_hbm.at[pk_cur.at[0]], xs_v, sxP).wait()
                pltpu.make_async_copy(
                    r2_hbm.at[pk_cur.at[2]], rr_v, srP).wait()
                if first_ok is None:
                    wait_sc(scQ)
                else:
                    @pl.when(first_ok)
                    def _w():
                        wait_sc(scQ)
                fire_rr(pk_next, rrQ_v, srQ)
                compute(pk_cur, xs_v, rr_v)
                pltpu.async_copy(rr_v, acc_sh.at[pk_cur.at[1]], scP,
                                 add=True)
                if fill_ok is None:
                    fire_pk(kk + 2, pk_fill, sp_fill)
                    fire_xs(pk_fill, xs_v, sxP)
                else:
                    @pl.when(fill_ok)
                    def _f():
                        fire_pk(kk + 2, pk_fill, sp_fill)
                        fire_xs(pk_fill, xs_v, sxP)

            run(k, pka0, spa0, xsa_v, rra_v, sxa, sra, sca,
                pka1, spa1, None, pkb0, rrb_v, srb, scb, first_ok=(k > 0))
            run(k + 1, pkb0, spb0, xsb_v, rrb_v, sxb, srb, scb,
                pkb1, spb1, None, pka1, rra_v, sra, sca)
            run(k + 2, pka1, spa1, xsa_v, rra_v, sxa, sra, sca,
                pka0, spa0, None, pkb1, rrb_v, srb, scb)
            run(k + 3, pkb1, spb1, xsb_v, rrb_v, sxb, srb, scb,
                pkb0, spb0, k + 5 < NCH, pka0, rra_v, sra, sca)

        kf = NCH - 1
        wait_pk(kf, pka0, spa0)
        pltpu.make_async_copy(x_hbm.at[pka0.at[0]], xsa_v, sxa).wait()
        pltpu.make_async_copy(r2_hbm.at[pka0.at[2]], rra_v, sra).wait()
        compute(pka0, xsa_v, rra_v)
        pltpu.async_copy(rra_v, acc_sh.at[pka0.at[1]], sca, add=True)
        wait_sc(scb)
        wait_sc(sca)

        plsc.subcore_barrier()
        pltpu.sync_copy(acc_sh.at[pl.ds(row0, 624)],
                        out_hbm.at[cid, pl.ds(row0, 624)])

        @pl.when(sid == 15)
        def _wtail():
            pltpu.sync_copy(acc_sh.at[pl.ds(9984, 16)],
                            out_hbm.at[cid, pl.ds(9984, 16)])

    return _pass1, _pass2


def kernel(x, edge_index, edge_type, rel_emb, res_att, W_ww, W_rel):
    pass1, pass2 = _sc_kernels()
    r2, rel_out = _prep(rel_emb, W_ww, W_rel)
    P = _pmat(x, r2)
    src = edge_index[0].reshape(NW, NCH, CH)
    dst = edge_index[1].reshape(NW, NCH, CH)
    et = edge_type.reshape(NW, NCH, CH)
    pidx = _pidx(dst, et)
    pk1 = jnp.stack([src, dst, pidx], axis=2)
    dp = pass1(x, P.reshape(N * REL), pk1)
    w = _softmax(dp)
    pk2 = jnp.stack(
        [src, dst, et, lax.bitcast_convert_type(w, jnp.int32)], axis=2)
    partials = pass2(x, r2, pk2)
    out = _final(partials)
    return (out, rel_out, res_att)


# speedup vs baseline: 1.1753x; 1.1753x over previous
import dataclasses
import functools

import jax
import jax.numpy as jnp
from jax import lax
from jax.experimental import pallas as pl
from jax.experimental.pallas import tpu as pltpu
from jax.experimental.pallas import tpu_sc as plsc

N = 10000
E = 320000
REL = 500
EH = 128

NW = 32
EPW = E // NW
CH = 80
NCH = EPW // CH


def _prep_body(rel_ref, ww_ref, wrel_ref, dst_ref, et_ref,
               r2_ref, ro_ref, pi_ref):
    r = rel_ref[...]
    dn = (((1,), (0,)), ((), ()))
    r2_ref[...] = lax.dot_general(r, ww_ref[...], dn,
                                  precision=lax.Precision.HIGHEST,
                                  preferred_element_type=jnp.float32)
    ro_ref[...] = lax.dot_general(r, wrel_ref[...], dn,
                                  precision=lax.Precision.HIGHEST,
                                  preferred_element_type=jnp.float32)
    pi_ref[...] = dst_ref[...] * REL + et_ref[...]


_prep = pl.pallas_call(
    _prep_body,
    out_shape=[jax.ShapeDtypeStruct((REL, EH), jnp.float32),
               jax.ShapeDtypeStruct((REL, EH), jnp.float32),
               jax.ShapeDtypeStruct((NW, NCH, CH), jnp.int32)],
)


def _pmat_body(x_ref, r2_ref, p_ref):
    dnt = (((1,), (1,)), ((), ()))
    p_ref[...] = lax.dot_general(x_ref[...], r2_ref[...], dnt,
                                 precision=lax.Precision.HIGHEST,
                                 preferred_element_type=jnp.float32)


_pmat = pl.pallas_call(
    _pmat_body,
    grid=(5,),
    in_specs=[pl.BlockSpec((N // 5, EH), lambda i: (i, 0)),
              pl.BlockSpec((REL, EH), lambda i: (0, 0))],
    out_specs=pl.BlockSpec((N // 5, REL), lambda i: (i, 0)),
    out_shape=jax.ShapeDtypeStruct((N, REL), jnp.float32),
)


def _softmax_body(dp_ref, w_ref):
    d = dp_ref[...]
    m = jnp.max(d)
    e = jnp.exp(d - m)
    w_ref[...] = e / jnp.sum(e)


_softmax = pl.pallas_call(
    _softmax_body,
    out_shape=jax.ShapeDtypeStruct((NW, NCH, CH), jnp.float32),
)


def _final_body(p_ref, out_ref):
    out_ref[...] = jnp.maximum(p_ref[0] + p_ref[1], 0.0)


_final = pl.pallas_call(
    _final_body,
    out_shape=jax.ShapeDtypeStruct((N, EH), jnp.float32),
)


@functools.cache
def _sc_kernels():
    mesh = plsc.VectorSubcoreMesh(core_axis_name="c", subcore_axis_name="s")
    cp = pltpu.CompilerParams()
    if "needs_layout_passes" in pltpu.CompilerParams.__dataclass_fields__:
        cp = dataclasses.replace(cp, needs_layout_passes=False)

    @functools.partial(
        pl.kernel,
        out_type=jax.ShapeDtypeStruct((NW, NCH, CH), jnp.float32),
        mesh=mesh,
        compiler_params=cp,
        scratch_types=[
            pltpu.VMEM((NCH, CH), jnp.int32),
            pltpu.VMEM((NCH, CH), jnp.int32),
            pltpu.VMEM((NCH, CH), jnp.int32),
            pltpu.VMEM((CH, EH), jnp.float32),
            pltpu.VMEM((CH, EH), jnp.float32),
            pltpu.VMEM((CH,), jnp.float32),
            pltpu.VMEM((CH, EH), jnp.float32),
            pltpu.VMEM((CH, EH), jnp.float32),
            pltpu.VMEM((CH,), jnp.float32),
            pltpu.VMEM((16, 17), jnp.float32),
            pltpu.VMEM((NCH, CH), jnp.float32),
            pltpu.SemaphoreType.DMA,
            pltpu.SemaphoreType.DMA,
            pltpu.SemaphoreType.DMA,
            pltpu.SemaphoreType.DMA,
            pltpu.SemaphoreType.DMA,
            pltpu.SemaphoreType.DMA,
        ],
    )
    def _pass1(x_hbm, pf_hbm, src_hbm, dst_hbm, pi_hbm, dp_hbm,
               src_v, dst_v, pi_v, xsa_v, xda_v, pva_v, xsb_v, xdb_v, pvb_v,
               tb_v, dp_v, sa0, sa1, sa2, sb0, sb1, sb2):
        cid = lax.axis_index("c")
        sid = lax.axis_index("s")
        wid = cid * 16 + sid
        pltpu.sync_copy(src_hbm.at[wid], src_v)
        pltpu.sync_copy(dst_hbm.at[wid], dst_v)
        pltpu.sync_copy(pi_hbm.at[wid], pi_v)
        lanes = lax.iota(jnp.int32, 16)

        def fire(k, xs_v, xd_v, pv_v, s0, s1, s2):
            c1 = pltpu.async_copy(x_hbm.at[src_v.at[k]], xs_v, s0)
            c2 = pltpu.async_copy(x_hbm.at[dst_v.at[k]], xd_v, s1)
            c3 = pltpu.async_copy(pf_hbm.at[pi_v.at[k]], pv_v, s2)
            return (c1, c2, c3)

        def compute(k, xs_v, xd_v, pv_v):
            @pl.loop(0, CH // 16)
            def _grp(g):
                e0 = g * 16
                for j in range(16):
                    b = e0 + j
                    acc = xs_v[b, pl.ds(0, 16)] * xd_v[b, pl.ds(0, 16)]
                    for c in range(1, 8):
                        sl = pl.ds(c * 16, 16)
                        acc = acc + xs_v[b, sl] * xd_v[b, sl]
                    tb_v[j, pl.ds(0, 16)] = acc
                dpv = pv_v[pl.ds(e0, 16)]
                for j2 in range(16):
                    jv = jnp.full((16,), j2, jnp.int32)
                    dpv = dpv + plsc.load_gather(tb_v, [lanes, jv])
                dp_v[k, pl.ds(e0, 16)] = dpv

        ca = fire(0, xsa_v, xda_v, pva_v, sa0, sa1, sa2)
        cb = fire(1, xsb_v, xdb_v, pvb_v, sb0, sb1, sb2)
        del ca, cb

        @pl.loop(0, NCH // 2)
        def _pair(i):
            k0 = i * 2
            k1 = k0 + 1
            pltpu.make_async_copy(x_hbm.at[src_v.at[k0]], xsa_v, sa0).wait()
            pltpu.make_async_copy(x_hbm.at[dst_v.at[k0]], xda_v, sa1).wait()
            pltpu.make_async_copy(pf_hbm.at[pi_v.at[k0]], pva_v, sa2).wait()
            compute(k0, xsa_v, xda_v, pva_v)
            fire(k0 + 2, xsa_v, xda_v, pva_v, sa0, sa1, sa2)
            pltpu.make_async_copy(x_hbm.at[src_v.at[k1]], xsb_v, sb0).wait()
            pltpu.make_async_copy(x_hbm.at[dst_v.at[k1]], xdb_v, sb1).wait()
            pltpu.make_async_copy(pf_hbm.at[pi_v.at[k1]], pvb_v, sb2).wait()
            compute(k1, xsb_v, xdb_v, pvb_v)

            @pl.when(k1 + 2 < NCH)
            def _fb():
                fire(k1 + 2, xsb_v, xdb_v, pvb_v, sb0, sb1, sb2)

        pltpu.make_async_copy(x_hbm.at[src_v.at[NCH - 1]], xsa_v, sa0).wait()
        pltpu.make_async_copy(x_hbm.at[dst_v.at[NCH - 1]], xda_v, sa1).wait()
        pltpu.make_async_copy(pf_hbm.at[pi_v.at[NCH - 1]], pva_v, sa2).wait()
        compute(NCH - 1, xsa_v, xda_v, pva_v)

        pltpu.sync_copy(dp_v, dp_hbm.at[wid])

    @functools.partial(
        pl.kernel,
        out_type=jax.ShapeDtypeStruct((2, N, EH), jnp.float32),
        mesh=mesh,
        compiler_params=cp,
        scratch_types=[
            pltpu.VMEM((4, CH), jnp.int32),
            pltpu.VMEM((CH, EH), jnp.float32),
            pltpu.VMEM((CH, EH), jnp.float32),
            pltpu.VMEM((4, CH), jnp.int32),
            pltpu.VMEM((CH, EH), jnp.float32),
            pltpu.VMEM((CH, EH), jnp.float32),
            pltpu.VMEM_SHARED((N, EH), jnp.float32),
            pltpu.SemaphoreType.DMA,
            pltpu.SemaphoreType.DMA,
            pltpu.SemaphoreType.DMA,
            pltpu.SemaphoreType.DMA,
        ],
    )
    def _pass2(x_hbm, r2_hbm, pk_hbm, out_hbm,
               pka_v, xsa_v, rra_v, pkb_v, xsb_v, rrb_v, acc_sh,
               sa0, sa1, sb0, sb1):
        cid = lax.axis_index("c")
        sid = lax.axis_index("s")
        wid = cid * 16 + sid
        zero16 = jnp.zeros((16,), jnp.float32)

        @pl.loop(0, CH)
        def _zrow(i):
            for c in range(8):
                xsa_v[i, pl.ds(c * 16, 16)] = zero16

        row0 = pl.multiple_of(sid * 624, 16)
        for q in range(7):
            pltpu.sync_copy(xsa_v, acc_sh.at[pl.ds(row0 + q * CH, CH)])
        pltpu.sync_copy(xsa_v.at[pl.ds(0, 64)],
                        acc_sh.at[pl.ds(row0 + 560, 64)])

        @pl.when(sid == 15)
        def _ztail():
            pltpu.sync_copy(xsa_v.at[pl.ds(0, 16)],
                            acc_sh.at[pl.ds(9984, 16)])

        plsc.subcore_barrier()

        def fire(k, pk_v, xs_v, rr_v, s0, s1):
            pltpu.sync_copy(pk_hbm.at[wid, k], pk_v)
            pltpu.async_copy(x_hbm.at[pk_v.at[0]], xs_v, s0)
            pltpu.async_copy(r2_hbm.at[pk_v.at[2]], rr_v, s1)

        def wait(pk_v, xs_v, rr_v, s0, s1):
            pltpu.make_async_copy(x_hbm.at[pk_v.at[0]], xs_v, s0).wait()
            pltpu.make_async_copy(r2_hbm.at[pk_v.at[2]], rr_v, s1).wait()

        def compute_scatter(pk_v, xs_v, rr_v):
            @pl.loop(0, CH // 16)
            def _grp(g):
                e0 = g * 16
                wg = plsc.bitcast(pk_v[3, pl.ds(e0, 16)], jnp.float32)
                for j in range(16):
                    b = e0 + j
                    wvec = jnp.full((16,), wg[j], jnp.float32)
                    for c in range(8):
                        sl = pl.ds(c * 16, 16)
                        xs_v[b, sl] = (xs_v[b, sl] + rr_v[b, sl]) * wvec

            pltpu.sync_copy(xs_v, acc_sh.at[pk_v.at[1]], add=True)

        fire(0, pka_v, xsa_v, rra_v, sa0, sa1)
        fire(1, pkb_v, xsb_v, rrb_v, sb0, sb1)

        @pl.loop(0, NCH // 2)
        def _pair(i):
            k0 = i * 2
            k1 = k0 + 1
            wait(pka_v, xsa_v, rra_v, sa0, sa1)
            compute_scatter(pka_v, xsa_v, rra_v)
            fire(k0 + 2, pka_v, xsa_v, rra_v, sa0, sa1)
            wait(pkb_v, xsb_v, rrb_v, sb0, sb1)
            compute_scatter(pkb_v, xsb_v, rrb_v)

            @pl.when(k1 + 2 < NCH)
            def _fb():
                fire(k1 + 2, pkb_v, xsb_v, rrb_v, sb0, sb1)

        wait(pka_v, xsa_v, rra_v, sa0, sa1)
        compute_scatter(pka_v, xsa_v, rra_v)

        plsc.subcore_barrier()
        pltpu.sync_copy(acc_sh.at[pl.ds(row0, 624)],
                        out_hbm.at[cid, pl.ds(row0, 624)])

        @pl.when(sid == 15)
        def _wtail():
            pltpu.sync_copy(acc_sh.at[pl.ds(9984, 16)],
                            out_hbm.at[cid, pl.ds(9984, 16)])

    return _pass1, _pass2


def kernel(x, edge_index, edge_type, rel_emb, res_att, W_ww, W_rel):
    pass1, pass2 = _sc_kernels()
    src = edge_index[0].reshape(NW, NCH, CH)
    dst = edge_index[1].reshape(NW, NCH, CH)
    et = edge_type.reshape(NW, NCH, CH)
    r2, rel_out, pidx = _prep(rel_emb, W_ww, W_rel, dst, et)
    P = _pmat(x, r2)
    dp = pass1(x, P.reshape(N * REL), src, dst, pidx)
    w = _softmax(dp)
    pk2 = jnp.stack(
        [src, dst, et, lax.bitcast_convert_type(w, jnp.int32)], axis=2)
    partials = pass2(x, r2, pk2)
    out = _final(partials)
    return (out, rel_out, res_att)


# speedup vs baseline: 1.1783x; 1.0026x over previous
import dataclasses
import functools

import jax
import jax.numpy as jnp
from jax import lax
from jax.experimental import pallas as pl
from jax.experimental.pallas import tpu as pltpu
from jax.experimental.pallas import tpu_sc as plsc

N = 10000
E = 320000
REL = 500
EH = 128

NW = 32
EPW = E // NW
CH = 80
NCH = EPW // CH


def _prep_body(rel_ref, ww_ref, wrel_ref, dst_ref, et_ref,
               r2_ref, ro_ref, pi_ref, r2p_ref):
    r = rel_ref[...]
    dn = (((1,), (0,)), ((), ()))
    r2 = lax.dot_general(r, ww_ref[...], dn,
                         precision=lax.Precision.HIGHEST,
                         preferred_element_type=jnp.float32)
    r2_ref[...] = r2
    ro_ref[...] = lax.dot_general(r, wrel_ref[...], dn,
                                  precision=lax.Precision.HIGHEST,
                                  preferred_element_type=jnp.float32)
    pi_ref[...] = dst_ref[...] * REL + et_ref[...]
    lob = lax.bitcast_convert_type(
        r2[:, :64].astype(jnp.bfloat16), jnp.uint16).astype(jnp.uint32)
    hib = lax.bitcast_convert_type(
        r2[:, 64:].astype(jnp.bfloat16), jnp.uint16).astype(jnp.uint32)
    r2p_ref[...] = lax.bitcast_convert_type(
        lob | (hib << jnp.uint32(16)), jnp.int32)


_prep = pl.pallas_call(
    _prep_body,
    out_shape=[jax.ShapeDtypeStruct((REL, EH), jnp.float32),
               jax.ShapeDtypeStruct((REL, EH), jnp.float32),
               jax.ShapeDtypeStruct((NW, NCH, CH), jnp.int32),
               jax.ShapeDtypeStruct((REL, 64), jnp.int32)],
)


def _pmat_body(x_ref, r2_ref, p_ref):
    dnt = (((1,), (1,)), ((), ()))
    p_ref[...] = lax.dot_general(x_ref[...], r2_ref[...], dnt,
                                 precision=lax.Precision.HIGHEST,
                                 preferred_element_type=jnp.float32)


_pmat = pl.pallas_call(
    _pmat_body,
    grid=(5,),
    in_specs=[pl.BlockSpec((N // 5, EH), lambda i: (i, 0)),
              pl.BlockSpec((REL, EH), lambda i: (0, 0))],
    out_specs=pl.BlockSpec((N // 5, REL), lambda i: (i, 0)),
    out_shape=jax.ShapeDtypeStruct((N, REL), jnp.float32),
)


def _softmax_body(dp_ref, w_ref):
    d = dp_ref[...]
    m = jnp.max(d)
    e = jnp.exp(d - m)
    w_ref[...] = e / jnp.sum(e)


_softmax = pl.pallas_call(
    _softmax_body,
    out_shape=jax.ShapeDtypeStruct((NW, NCH, CH), jnp.float32),
)


def _final_body(p_ref, out_ref):
    out_ref[...] = jnp.maximum(p_ref[0] + p_ref[1], 0.0)


_final = pl.pallas_call(
    _final_body,
    out_shape=jax.ShapeDtypeStruct((N, EH), jnp.float32),
)


@functools.cache
def _sc_kernels():
    mesh = plsc.VectorSubcoreMesh(core_axis_name="c", subcore_axis_name="s")
    cp = pltpu.CompilerParams()
    if "needs_layout_passes" in pltpu.CompilerParams.__dataclass_fields__:
        cp = dataclasses.replace(cp, needs_layout_passes=False)
    cp2 = dataclasses.replace(cp, use_tc_tiling_on_sc=False)

    @functools.partial(
        pl.kernel,
        out_type=jax.ShapeDtypeStruct((NW, NCH, CH), jnp.float32),
        mesh=mesh,
        compiler_params=cp,
        scratch_types=[
            pltpu.VMEM((NCH, CH), jnp.int32),
            pltpu.VMEM((NCH, CH), jnp.int32),
            pltpu.VMEM((NCH, CH), jnp.int32),
            pltpu.VMEM((CH, EH), jnp.float32),
            pltpu.VMEM((CH, EH), jnp.float32),
            pltpu.VMEM((CH,), jnp.float32),
            pltpu.VMEM((CH, EH), jnp.float32),
            pltpu.VMEM((CH, EH), jnp.float32),
            pltpu.VMEM((CH,), jnp.float32),
            pltpu.VMEM((16, 17), jnp.float32),
            pltpu.VMEM((NCH, CH), jnp.float32),
            pltpu.SemaphoreType.DMA,
            pltpu.SemaphoreType.DMA,
            pltpu.SemaphoreType.DMA,
            pltpu.SemaphoreType.DMA,
            pltpu.SemaphoreType.DMA,
            pltpu.SemaphoreType.DMA,
        ],
    )
    def _pass1(x_hbm, pf_hbm, src_hbm, dst_hbm, pi_hbm, dp_hbm,
               src_v, dst_v, pi_v, xsa_v, xda_v, pva_v, xsb_v, xdb_v, pvb_v,
               tb_v, dp_v, sa0, sa1, sa2, sb0, sb1, sb2):
        cid = lax.axis_index("c")
        sid = lax.axis_index("s")
        wid = cid * 16 + sid
        pltpu.sync_copy(src_hbm.at[wid], src_v)
        pltpu.sync_copy(dst_hbm.at[wid], dst_v)
        pltpu.sync_copy(pi_hbm.at[wid], pi_v)
        lanes = lax.iota(jnp.int32, 16)

        def fire(k, xs_v, xd_v, pv_v, s0, s1, s2):
            c1 = pltpu.async_copy(x_hbm.at[src_v.at[k]], xs_v, s0)
            c2 = pltpu.async_copy(x_hbm.at[dst_v.at[k]], xd_v, s1)
            c3 = pltpu.async_copy(pf_hbm.at[pi_v.at[k]], pv_v, s2)
            return (c1, c2, c3)

        def compute(k, xs_v, xd_v, pv_v):
            @pl.loop(0, CH // 16)
            def _grp(g):
                e0 = g * 16
                for j in range(16):
                    b = e0 + j
                    acc = xs_v[b, pl.ds(0, 16)] * xd_v[b, pl.ds(0, 16)]
                    for c in range(1, 8):
                        sl = pl.ds(c * 16, 16)
                        acc = acc + xs_v[b, sl] * xd_v[b, sl]
                    tb_v[j, pl.ds(0, 16)] = acc
                dpv = pv_v[pl.ds(e0, 16)]
                for j2 in range(16):
                    jv = jnp.full((16,), j2, jnp.int32)
                    dpv = dpv + plsc.load_gather(tb_v, [lanes, jv])
                dp_v[k, pl.ds(e0, 16)] = dpv

        ca = fire(0, xsa_v, xda_v, pva_v, sa0, sa1, sa2)
        cb = fire(1, xsb_v, xdb_v, pvb_v, sb0, sb1, sb2)
        del ca, cb

        @pl.loop(0, NCH // 2)
        def _pair(i):
            k0 = i * 2
            k1 = k0 + 1
            pltpu.make_async_copy(x_hbm.at[src_v.at[k0]], xsa_v, sa0).wait()
            pltpu.make_async_copy(x_hbm.at[dst_v.at[k0]], xda_v, sa1).wait()
            pltpu.make_async_copy(pf_hbm.at[pi_v.at[k0]], pva_v, sa2).wait()
            compute(k0, xsa_v, xda_v, pva_v)
            fire(k0 + 2, xsa_v, xda_v, pva_v, sa0, sa1, sa2)
            pltpu.make_async_copy(x_hbm.at[src_v.at[k1]], xsb_v, sb0).wait()
            pltpu.make_async_copy(x_hbm.at[dst_v.at[k1]], xdb_v, sb1).wait()
            pltpu.make_async_copy(pf_hbm.at[pi_v.at[k1]], pvb_v, sb2).wait()
            compute(k1, xsb_v, xdb_v, pvb_v)

            @pl.when(k1 + 2 < NCH)
            def _fb():
                fire(k1 + 2, xsb_v, xdb_v, pvb_v, sb0, sb1, sb2)

        pltpu.make_async_copy(x_hbm.at[src_v.at[NCH - 1]], xsa_v, sa0).wait()
        pltpu.make_async_copy(x_hbm.at[dst_v.at[NCH - 1]], xda_v, sa1).wait()
        pltpu.make_async_copy(pf_hbm.at[pi_v.at[NCH - 1]], pva_v, sa2).wait()
        compute(NCH - 1, xsa_v, xda_v, pva_v)

        pltpu.sync_copy(dp_v, dp_hbm.at[wid])

    @functools.partial(
        pl.kernel,
        out_type=jax.ShapeDtypeStruct((2, N, EH), jnp.float32),
        mesh=mesh,
        compiler_params=cp2,
        scratch_types=[
            pltpu.VMEM((4, CH), jnp.int32),
            pltpu.VMEM((CH, EH), jnp.float32),
            pltpu.VMEM((CH, 64), jnp.int32),
            pltpu.VMEM((4, CH), jnp.int32),
            pltpu.VMEM((CH, EH), jnp.float32),
            pltpu.VMEM((CH, 64), jnp.int32),
            pltpu.VMEM_SHARED((N, EH), jnp.float32),
            pltpu.SemaphoreType.DMA,
            pltpu.SemaphoreType.DMA,
            pltpu.SemaphoreType.DMA,
            pltpu.SemaphoreType.DMA,
        ],
    )
    def _pass2(x_hbm, r2_hbm, pk_hbm, out_hbm,
               pka_v, xsa_v, rra_v, pkb_v, xsb_v, rrb_v, acc_sh,
               sa0, sa1, sb0, sb1):
        cid = lax.axis_index("c")
        sid = lax.axis_index("s")
        wid = cid * 16 + sid
        zero16 = jnp.zeros((16,), jnp.float32)

        @pl.loop(0, CH)
        def _zrow(i):
            for c in range(8):
                xsa_v[i, pl.ds(c * 16, 16)] = zero16

        row0 = pl.multiple_of(sid * 624, 16)
        for q in range(7):
            pltpu.sync_copy(xsa_v, acc_sh.at[pl.ds(row0 + q * CH, CH)])
        pltpu.sync_copy(xsa_v.at[pl.ds(0, 64)],
                        acc_sh.at[pl.ds(row0 + 560, 64)])

        @pl.when(sid == 15)
        def _ztail():
            pltpu.sync_copy(xsa_v.at[pl.ds(0, 16)],
                            acc_sh.at[pl.ds(9984, 16)])

        plsc.subcore_barrier()

        def fire(k, pk_v, xs_v, rr_v, s0, s1):
            pltpu.sync_copy(pk_hbm.at[wid, k], pk_v)
            pltpu.async_copy(x_hbm.at[pk_v.at[0]], xs_v, s0)
            pltpu.async_copy(r2_hbm.at[pk_v.at[2]], rr_v, s1)

        def wait(pk_v, xs_v, rr_v, s0, s1):
            pltpu.make_async_copy(x_hbm.at[pk_v.at[0]], xs_v, s0).wait()
            pltpu.make_async_copy(r2_hbm.at[pk_v.at[2]], rr_v, s1).wait()

        himask = jnp.full((16,), -65536, jnp.int32)

        def compute_scatter(pk_v, xs_v, rr_v):
            @pl.loop(0, CH // 16)
            def _grp(g):
                e0 = g * 16
                wg = plsc.bitcast(pk_v[3, pl.ds(e0, 16)], jnp.float32)
                for j in range(16):
                    b = e0 + j
                    wvec = jnp.full((16,), wg[j], jnp.float32)
                    for cw in range(4):
                        wv = rr_v[b, pl.ds(cw * 16, 16)]
                        lo = plsc.bitcast(lax.shift_left(wv, 16),
                                          jnp.float32)
                        hi = plsc.bitcast(wv & himask, jnp.float32)
                        sl = pl.ds(cw * 16, 16)
                        sh = pl.ds((cw + 4) * 16, 16)
                        xs_v[b, sl] = (xs_v[b, sl] + lo) * wvec
                        xs_v[b, sh] = (xs_v[b, sh] + hi) * wvec

            pltpu.sync_copy(xs_v, acc_sh.at[pk_v.at[1]], add=True)

        fire(0, pka_v, xsa_v, rra_v, sa0, sa1)
        fire(1, pkb_v, xsb_v, rrb_v, sb0, sb1)

        @pl.loop(0, NCH // 2)
        def _pair(i):
            k0 = i * 2
            k1 = k0 + 1
            wait(pka_v, xsa_v, rra_v, sa0, sa1)
            compute_scatter(pka_v, xsa_v, rra_v)
            fire(k0 + 2, pka_v, xsa_v, rra_v, sa0, sa1)
            wait(pkb_v, xsb_v, rrb_v, sb0, sb1)
            compute_scatter(pkb_v, xsb_v, rrb_v)

            @pl.when(k1 + 2 < NCH)
            def _fb():
                fire(k1 + 2, pkb_v, xsb_v, rrb_v, sb0, sb1)

        wait(pka_v, xsa_v, rra_v, sa0, sa1)
        compute_scatter(pka_v, xsa_v, rra_v)

        plsc.subcore_barrier()
        pltpu.sync_copy(acc_sh.at[pl.ds(row0, 624)],
                        out_hbm.at[cid, pl.ds(row0, 624)])

        @pl.when(sid == 15)
        def _wtail():
            pltpu.sync_copy(acc_sh.at[pl.ds(9984, 16)],
                            out_hbm.at[cid, pl.ds(9984, 16)])

    return _pass1, _pass2


def kernel(x, edge_index, edge_type, rel_emb, res_att, W_ww, W_rel):
    pass1, pass2 = _sc_kernels()
    src = edge_index[0].reshape(NW, NCH, CH)
    dst = edge_index[1].reshape(NW, NCH, CH)
    et = edge_type.reshape(NW, NCH, CH)
    r2, rel_out, pidx, r2p = _prep(rel_emb, W_ww, W_rel, dst, et)
    P = _pmat(x, r2)
    dp = pass1(x, P.reshape(N * REL), src, dst, pidx)
    w = _softmax(dp)
    pk2 = jnp.stack(
        [src, dst, et, lax.bitcast_convert_type(w, jnp.int32)], axis=2)
    partials = pass2(x, r2p, pk2)
    out = _final(partials)
    return (out, rel_out, res_att)
